# Initial kernel scaffold; baseline (speedup 1.0000x reference)
import jax
import jax.numpy as jnp
from jax.experimental import pallas as pl


def kernel(seq_len, rel_pos_emb):
    raise NotImplementedError("write your pallas kernel here")


# speedup vs baseline: 5.1682x; 5.1682x over previous
import jax
import jax.numpy as jnp
from jax.experimental import pallas as pl
from jax.experimental.pallas import tpu as pltpu


def kernel(seq_len, rel_pos_emb):
    n_emb, d = rel_pos_emb.shape
    s = (n_emb + 1) // 2
    n_pad = n_emb + 1
    base = n_emb - seq_len

    def body(base_ref, emb_ref, out_ref, femb8_ref):
        i = pl.program_id(0)

        @pl.when(i == 0)
        def _():
            r = jax.lax.broadcasted_iota(jnp.int32, (n_pad, n_emb), 0)
            c = jax.lax.broadcasted_iota(jnp.int32, (n_pad, n_emb), 1)
            perm = (r + c == n_emb - 1).astype(emb_ref.dtype)
            femb = jnp.dot(perm, emb_ref[...], preferred_element_type=jnp.float32,
                           precision=jax.lax.Precision.HIGHEST)
            for p in range(8):
                femb8_ref[p] = pltpu.roll(femb, (n_pad - p) % n_pad, 0)

        start = base_ref[0] - i
        p = jax.lax.rem(start, 8)
        a = pl.multiple_of(start - p, 8)
        out_ref[0] = femb8_ref[p, pl.ds(a, s), :]

    out = pl.pallas_call(
        body,
        grid_spec=pltpu.PrefetchScalarGridSpec(
            num_scalar_prefetch=1,
            grid=(s,),
            in_specs=[pl.BlockSpec((n_emb, d), lambda i, base: (0, 0))],
            out_specs=pl.BlockSpec((1, s, d), lambda i, base: (i, 0, 0)),
            scratch_shapes=[pltpu.VMEM((8, n_pad, d), rel_pos_emb.dtype)],
        ),
        out_shape=jax.ShapeDtypeStruct((s, s, d), rel_pos_emb.dtype),
    )(jnp.asarray(base, jnp.int32).reshape(1), rel_pos_emb)
    return out


# speedup vs baseline: 10.3925x; 2.0109x over previous
import jax
import jax.numpy as jnp
from jax.experimental import pallas as pl
from jax.experimental.pallas import tpu as pltpu

_NBUF = 4


def kernel(seq_len, rel_pos_emb):
    n_emb, d = rel_pos_emb.shape
    s = (n_emb + 1) // 2
    n_pad = n_emb + 1
    base = n_emb - seq_len

    def body(base_ref, emb_ref, out_ref, femb8_ref, sems):
        i = pl.program_id(0)

        @pl.when(i == 0)
        def _():
            r = jax.lax.broadcasted_iota(jnp.int32, (n_pad, n_emb), 0)
            c = jax.lax.broadcasted_iota(jnp.int32, (n_pad, n_emb), 1)
            perm = (r + c == n_emb - 1).astype(emb_ref.dtype)
            femb = jnp.dot(perm, emb_ref[...], preferred_element_type=jnp.float32,
                           precision=jax.lax.Precision.HIGHEST)
            for p in range(8):
                femb8_ref[p] = pltpu.roll(femb, (n_pad - p) % n_pad, 0)

        start = base_ref[0] - i
        p = jax.lax.rem(start, 8)
        a = pl.multiple_of(start - p, 8)

        @pl.when(i >= _NBUF)
        def _():
            pltpu.make_async_copy(
                femb8_ref.at[0, pl.ds(0, s), :], out_ref.at[0], sems.at[i % _NBUF]
            ).wait()

        pltpu.make_async_copy(
            femb8_ref.at[p, pl.ds(a, s), :], out_ref.at[i], sems.at[i % _NBUF]
        ).start()

        @pl.when(i == s - 1)
        def _():
            for k in range(_NBUF):
                pltpu.make_async_copy(
                    femb8_ref.at[0, pl.ds(0, s), :], out_ref.at[0], sems.at[k]
                ).wait()

    out = pl.pallas_call(
        body,
        grid_spec=pltpu.PrefetchScalarGridSpec(
            num_scalar_prefetch=1,
            grid=(s,),
            in_specs=[pl.BlockSpec((n_emb, d), lambda i, base: (0, 0))],
            out_specs=pl.BlockSpec(memory_space=pl.ANY),
            scratch_shapes=[
                pltpu.VMEM((8, n_pad, d), rel_pos_emb.dtype),
                pltpu.SemaphoreType.DMA((_NBUF,)),
            ],
        ),
        out_shape=jax.ShapeDtypeStruct((s, s, d), rel_pos_emb.dtype),
    )(jnp.asarray(base, jnp.int32).reshape(1), rel_pos_emb)
    return out


# speedup vs baseline: 13.3208x; 1.2818x over previous
import jax
import jax.numpy as jnp
from jax.experimental import pallas as pl
from jax.experimental.pallas import tpu as pltpu

_NBUF = 16


def kernel(seq_len, rel_pos_emb):
    n_emb, d = rel_pos_emb.shape
    s = (n_emb + 1) // 2
    n_pad = n_emb + 1
    base = n_emb - seq_len

    def body(base_ref, emb_ref, out_ref, femb8_ref, sems):
        i = pl.program_id(0)

        @pl.when(i == 0)
        def _():
            r = jax.lax.broadcasted_iota(jnp.int32, (n_pad, n_emb), 0)
            c = jax.lax.broadcasted_iota(jnp.int32, (n_pad, n_emb), 1)
            perm = (r + c == n_emb - 1).astype(emb_ref.dtype)
            femb = jnp.dot(perm, emb_ref[...], preferred_element_type=jnp.float32,
                           precision=jax.lax.Precision.HIGHEST)
            for p in range(8):
                femb8_ref[p] = pltpu.roll(femb, (n_pad - p) % n_pad, 0)

        start = base_ref[0] - i
        p = jax.lax.rem(start, 8)
        a = pl.multiple_of(start - p, 8)

        @pl.when(i >= _NBUF)
        def _():
            pltpu.make_async_copy(
                femb8_ref.at[0, pl.ds(0, s), :], out_ref.at[0], sems.at[i % _NBUF]
            ).wait()

        pltpu.make_async_copy(
            femb8_ref.at[p, pl.ds(a, s), :], out_ref.at[i], sems.at[i % _NBUF]
        ).start()

        @pl.when(i == s - 1)
        def _():
            for k in range(_NBUF):
                pltpu.make_async_copy(
                    femb8_ref.at[0, pl.ds(0, s), :], out_ref.at[0], sems.at[k]
                ).wait()

    out = pl.pallas_call(
        body,
        grid_spec=pltpu.PrefetchScalarGridSpec(
            num_scalar_prefetch=1,
            grid=(s,),
            in_specs=[pl.BlockSpec((n_emb, d), lambda i, base: (0, 0))],
            out_specs=pl.BlockSpec(memory_space=pl.ANY),
            scratch_shapes=[
                pltpu.VMEM((8, n_pad, d), rel_pos_emb.dtype),
                pltpu.SemaphoreType.DMA((_NBUF,)),
            ],
        ),
        out_shape=jax.ShapeDtypeStruct((s, s, d), rel_pos_emb.dtype),
    )(jnp.asarray(base, jnp.int32).reshape(1), rel_pos_emb)
    return out
